# Initial kernel scaffold; baseline (speedup 1.0000x reference)
import jax
import jax.numpy as jnp
from jax.experimental import pallas as pl


def kernel(x, edge_index, W, b):
    raise NotImplementedError("write your pallas kernel here")


# speedup vs baseline: 7.1503x; 7.1503x over previous
import functools

import jax
import jax.numpy as jnp
from jax import lax
from jax.experimental import pallas as pl
from jax.experimental.pallas import tpu as pltpu
from jax.experimental.pallas import tpu_sc as plsc

N = 10000
E = 320000
D = 128

NTILE = 32
NP = 10240
EP = 327680
EPW = EP // NTILE
BLK = 128
NBLK = EPW // BLK
RPT = NP // 16

_MESH = plsc.VectorSubcoreMesh(core_axis_name="c", subcore_axis_name="s")


def _sc_deg_body(dst_ref, ones_ref, zeros_ref, deg0_ref, deg1_ref,
                 ones_v, idx_v, deg_sh):
    c = lax.axis_index("c")
    s = lax.axis_index("s")
    pltpu.sync_copy(zeros_ref.at[pl.ds(s * RPT, RPT)],
                    deg_sh.at[pl.ds(s * RPT, RPT)])
    pltpu.sync_copy(ones_ref, ones_v)
    plsc.subcore_barrier()
    base = (c * 16 + s) * EPW

    def step(i, carry):
        off = base + i * BLK
        pltpu.sync_copy(dst_ref.at[pl.ds(off, BLK)], idx_v)
        pltpu.sync_copy(ones_v, deg_sh.at[idx_v], add=True)
        return carry

    lax.fori_loop(0, NBLK, step, 0)
    plsc.subcore_barrier()

    @pl.when(c == 0)
    def _():
        pltpu.sync_copy(deg_sh.at[pl.ds(s * RPT, RPT)],
                        deg0_ref.at[pl.ds(s * RPT, RPT)])

    @pl.when(c == 1)
    def _():
        pltpu.sync_copy(deg_sh.at[pl.ds(s * RPT, RPT)],
                        deg1_ref.at[pl.ds(s * RPT, RPT)])


_sc_deg = pl.kernel(
    _sc_deg_body,
    out_type=(jax.ShapeDtypeStruct((NP,), jnp.float32),
              jax.ShapeDtypeStruct((NP,), jnp.float32)),
    mesh=_MESH,
    scratch_types=[
        pltpu.VMEM((BLK,), jnp.float32),
        pltpu.VMEM((BLK,), jnp.int32),
        pltpu.VMEM_SHARED((NP,), jnp.float32),
    ],
)


def _sc_hop_body(h_ref, zeros_ref, src_ref, dst_ref, pa_ref, pb_ref,
                 src_v, dst_v, rows_v, acc_sh, sem):
    c = lax.axis_index("c")
    s = lax.axis_index("s")
    pltpu.sync_copy(zeros_ref.at[pl.ds(s * RPT, RPT)],
                    acc_sh.at[pl.ds(s * RPT, RPT)])
    plsc.subcore_barrier()
    base = (c * 16 + s) * EPW

    def step(i, carry):
        off = base + i * BLK
        pltpu.sync_copy(src_ref.at[pl.ds(off, BLK)], src_v)
        pltpu.sync_copy(dst_ref.at[pl.ds(off, BLK)], dst_v)
        pltpu.async_copy(h_ref.at[src_v], rows_v, sem).wait()
        pltpu.sync_copy(rows_v, acc_sh.at[dst_v], add=True)
        return carry

    lax.fori_loop(0, NBLK, step, 0)
    plsc.subcore_barrier()

    @pl.when(c == 0)
    def _():
        pltpu.sync_copy(acc_sh.at[pl.ds(s * RPT, RPT)],
                        pa_ref.at[pl.ds(s * RPT, RPT)])

    @pl.when(c == 1)
    def _():
        pltpu.sync_copy(acc_sh.at[pl.ds(s * RPT, RPT)],
                        pb_ref.at[pl.ds(s * RPT, RPT)])


_sc_hop = pl.kernel(
    _sc_hop_body,
    out_type=(jax.ShapeDtypeStruct((NP, D), jnp.float32),
              jax.ShapeDtypeStruct((NP, D), jnp.float32)),
    mesh=_MESH,
    scratch_types=[
        pltpu.VMEM((BLK,), jnp.int32),
        pltpu.VMEM((BLK,), jnp.int32),
        pltpu.VMEM((BLK, D), jnp.float32),
        pltpu.VMEM_SHARED((NP, D), jnp.float32),
        pltpu.SemaphoreType.DMA,
    ],
)


R = 1024
G = NP // R


def _tc_scale_in_body(d0_ref, d1_ref, x_ref, y_ref):
    deg = d0_ref[...] + d1_ref[...] + 1.0
    y_ref[...] = x_ref[...] * lax.rsqrt(deg)


_tc_scale_in = pl.pallas_call(
    _tc_scale_in_body,
    grid=(G,),
    in_specs=[pl.BlockSpec((R, 1), lambda i: (i, 0)),
              pl.BlockSpec((R, 1), lambda i: (i, 0)),
              pl.BlockSpec((R, D), lambda i: (i, 0))],
    out_specs=pl.BlockSpec((R, D), lambda i: (i, 0)),
    out_shape=jax.ShapeDtypeStruct((NP, D), jnp.float32),
)


def _tc_mid_body(d0_ref, d1_ref, pa_ref, pb_ref, y_ref, h1_ref):
    deg = d0_ref[...] + d1_ref[...] + 1.0
    h1_ref[...] = (pa_ref[...] + pb_ref[...] + y_ref[...]) / deg


_tc_mid = pl.pallas_call(
    _tc_mid_body,
    grid=(G,),
    in_specs=[pl.BlockSpec((R, 1), lambda i: (i, 0)),
              pl.BlockSpec((R, 1), lambda i: (i, 0)),
              pl.BlockSpec((R, D), lambda i: (i, 0)),
              pl.BlockSpec((R, D), lambda i: (i, 0)),
              pl.BlockSpec((R, D), lambda i: (i, 0))],
    out_specs=pl.BlockSpec((R, D), lambda i: (i, 0)),
    out_shape=jax.ShapeDtypeStruct((NP, D), jnp.float32),
)


def _tc_final_body(d0_ref, d1_ref, pa_ref, pb_ref, h1_ref, w_ref, b_ref, o_ref):
    deg = d0_ref[...] + d1_ref[...] + 1.0
    h2 = (pa_ref[...] + pb_ref[...] + h1_ref[...]) * lax.rsqrt(deg)
    o = jnp.dot(h2, w_ref[...], preferred_element_type=jnp.float32) + b_ref[...]
    m = jnp.max(o, axis=-1, keepdims=True)
    e = o - m
    o_ref[...] = e - jnp.log(jnp.sum(jnp.exp(e), axis=-1, keepdims=True))


_tc_final = pl.pallas_call(
    _tc_final_body,
    grid=(G,),
    in_specs=[pl.BlockSpec((R, 1), lambda i: (i, 0)),
              pl.BlockSpec((R, 1), lambda i: (i, 0)),
              pl.BlockSpec((R, D), lambda i: (i, 0)),
              pl.BlockSpec((R, D), lambda i: (i, 0)),
              pl.BlockSpec((R, D), lambda i: (i, 0)),
              pl.BlockSpec((D, D), lambda i: (0, 0)),
              pl.BlockSpec((1, D), lambda i: (0, 0))],
    out_specs=pl.BlockSpec((R, D), lambda i: (i, 0)),
    out_shape=jax.ShapeDtypeStruct((NP, D), jnp.float32),
)


def kernel(x, edge_index, W, b):
    x_pad = jnp.pad(x, ((0, NP - N), (0, 0)))
    fill = jnp.full((EP - E,), NP - 1, dtype=jnp.int32)
    src = jnp.concatenate([edge_index[0].astype(jnp.int32), fill])
    dst = jnp.concatenate([edge_index[1].astype(jnp.int32), fill])
    zeros2d = jnp.zeros((NP, D), jnp.float32)
    zeros1d = jnp.zeros((NP,), jnp.float32)
    ones_blk = jnp.ones((BLK,), jnp.float32)

    deg0, deg1 = _sc_deg(dst, ones_blk, zeros1d)
    d0 = deg0.reshape(NP, 1)
    d1 = deg1.reshape(NP, 1)
    y = _tc_scale_in(d0, d1, x_pad)
    p1a, p1b = _sc_hop(y, zeros2d, src, dst)
    h1 = _tc_mid(d0, d1, p1a, p1b, y)
    p2a, p2b = _sc_hop(h1, zeros2d, src, dst)
    out = _tc_final(d0, d1, p2a, p2b, h1, W, b.reshape(1, D))
    return out[:N]


# speedup vs baseline: 15.7451x; 2.2020x over previous
import functools

import jax
import jax.numpy as jnp
from jax import lax
from jax.experimental import pallas as pl
from jax.experimental.pallas import tpu as pltpu
from jax.experimental.pallas import tpu_sc as plsc

N = 10000
E = 320000
D = 128

NTILE = 32
NP = 10240
EP = 327680
EPW = EP // NTILE
BLK = 128
NBLK = EPW // BLK
RPT = NP // 16

_MESH = plsc.VectorSubcoreMesh(core_axis_name="c", subcore_axis_name="s")


def _sc_deg_body(dst_ref, ones_ref, zeros_ref, deg0_ref, deg1_ref,
                 ones_v, idx_v, deg_sh):
    c = lax.axis_index("c")
    s = lax.axis_index("s")
    pltpu.sync_copy(zeros_ref.at[pl.ds(s * RPT, RPT)],
                    deg_sh.at[pl.ds(s * RPT, RPT)])
    pltpu.sync_copy(ones_ref, ones_v)
    plsc.subcore_barrier()
    base = (c * 16 + s) * EPW

    def step(i, carry):
        off = base + i * BLK
        pltpu.sync_copy(dst_ref.at[pl.ds(off, BLK)], idx_v)
        pltpu.sync_copy(ones_v, deg_sh.at[idx_v], add=True)
        return carry

    lax.fori_loop(0, NBLK, step, 0)
    plsc.subcore_barrier()

    @pl.when(c == 0)
    def _():
        pltpu.sync_copy(deg_sh.at[pl.ds(s * RPT, RPT)],
                        deg0_ref.at[pl.ds(s * RPT, RPT)])

    @pl.when(c == 1)
    def _():
        pltpu.sync_copy(deg_sh.at[pl.ds(s * RPT, RPT)],
                        deg1_ref.at[pl.ds(s * RPT, RPT)])


_sc_deg = pl.kernel(
    _sc_deg_body,
    out_type=(jax.ShapeDtypeStruct((NP,), jnp.float32),
              jax.ShapeDtypeStruct((NP,), jnp.float32)),
    mesh=_MESH,
    scratch_types=[
        pltpu.VMEM((BLK,), jnp.float32),
        pltpu.VMEM((BLK,), jnp.int32),
        pltpu.VMEM_SHARED((NP,), jnp.float32),
    ],
)


def _sc_hop_body(h_ref, zeros_ref, src_ref, dst_ref, pa_ref, pb_ref,
                 src_v, dst_v, rows_v, acc_sh, sem):
    c = lax.axis_index("c")
    s = lax.axis_index("s")
    pltpu.sync_copy(zeros_ref.at[pl.ds(s * RPT, RPT)],
                    acc_sh.at[pl.ds(s * RPT, RPT)])
    plsc.subcore_barrier()
    base = (c * 16 + s) * EPW

    def step(i, carry):
        off = base + i * BLK
        pltpu.sync_copy(src_ref.at[pl.ds(off, BLK)], src_v)
        pltpu.sync_copy(dst_ref.at[pl.ds(off, BLK)], dst_v)
        pltpu.async_copy(h_ref.at[src_v], rows_v, sem).wait()
        pltpu.sync_copy(rows_v, acc_sh.at[dst_v], add=True)
        return carry

    lax.fori_loop(0, NBLK, step, 0)
    plsc.subcore_barrier()

    @pl.when(c == 0)
    def _():
        pltpu.sync_copy(acc_sh.at[pl.ds(s * RPT, RPT)],
                        pa_ref.at[pl.ds(s * RPT, RPT)])

    @pl.when(c == 1)
    def _():
        pltpu.sync_copy(acc_sh.at[pl.ds(s * RPT, RPT)],
                        pb_ref.at[pl.ds(s * RPT, RPT)])


_sc_hop = pl.kernel(
    _sc_hop_body,
    out_type=(jax.ShapeDtypeStruct((NP, D), jnp.float32),
              jax.ShapeDtypeStruct((NP, D), jnp.float32)),
    mesh=_MESH,
    scratch_types=[
        pltpu.VMEM((BLK,), jnp.int32),
        pltpu.VMEM((BLK,), jnp.int32),
        pltpu.VMEM((BLK, D), jnp.float32),
        pltpu.VMEM_SHARED((NP, D), jnp.float32),
        pltpu.SemaphoreType.DMA,
    ],
)


R = 1024
G = NP // R


def _tc_scale_in_body(d0_ref, d1_ref, x_ref, y_ref):
    deg = d0_ref[...] + d1_ref[...] + 1.0
    y_ref[...] = x_ref[...] * lax.rsqrt(deg)


_tc_scale_in = pl.pallas_call(
    _tc_scale_in_body,
    grid=(G,),
    in_specs=[pl.BlockSpec((R, 1), lambda i: (i, 0)),
              pl.BlockSpec((R, 1), lambda i: (i, 0)),
              pl.BlockSpec((R, D), lambda i: (i, 0))],
    out_specs=pl.BlockSpec((R, D), lambda i: (i, 0)),
    out_shape=jax.ShapeDtypeStruct((NP, D), jnp.float32),
)


def _tc_mid_body(d0_ref, d1_ref, pa_ref, pb_ref, y_ref, h1_ref):
    deg = d0_ref[...] + d1_ref[...] + 1.0
    h1_ref[...] = (pa_ref[...] + pb_ref[...] + y_ref[...]) / deg


_tc_mid = pl.pallas_call(
    _tc_mid_body,
    grid=(G,),
    in_specs=[pl.BlockSpec((R, 1), lambda i: (i, 0)),
              pl.BlockSpec((R, 1), lambda i: (i, 0)),
              pl.BlockSpec((R, D), lambda i: (i, 0)),
              pl.BlockSpec((R, D), lambda i: (i, 0)),
              pl.BlockSpec((R, D), lambda i: (i, 0))],
    out_specs=pl.BlockSpec((R, D), lambda i: (i, 0)),
    out_shape=jax.ShapeDtypeStruct((NP, D), jnp.float32),
)


def _tc_final_body(d0_ref, d1_ref, pa_ref, pb_ref, h1_ref, w_ref, b_ref, o_ref):
    deg = d0_ref[...] + d1_ref[...] + 1.0
    h2 = (pa_ref[...] + pb_ref[...] + h1_ref[...]) * lax.rsqrt(deg)
    o = jnp.dot(h2, w_ref[...], preferred_element_type=jnp.float32) + b_ref[...]
    m = jnp.max(o, axis=-1, keepdims=True)
    e = o - m
    o_ref[...] = e - jnp.log(jnp.sum(jnp.exp(e), axis=-1, keepdims=True))


_tc_final = pl.pallas_call(
    _tc_final_body,
    grid=(G,),
    in_specs=[pl.BlockSpec((R, 1), lambda i: (i, 0)),
              pl.BlockSpec((R, 1), lambda i: (i, 0)),
              pl.BlockSpec((R, D), lambda i: (i, 0)),
              pl.BlockSpec((R, D), lambda i: (i, 0)),
              pl.BlockSpec((R, D), lambda i: (i, 0)),
              pl.BlockSpec((D, D), lambda i: (0, 0)),
              pl.BlockSpec((1, D), lambda i: (0, 0))],
    out_specs=pl.BlockSpec((R, D), lambda i: (i, 0)),
    out_shape=jax.ShapeDtypeStruct((NP, D), jnp.float32),
)


def kernel(x, edge_index, W, b):
    x_pad = jnp.pad(x, ((0, NP - N), (0, 0)))
    fill = N + jax.lax.rem(jnp.arange(EP - E, dtype=jnp.int32),
                           jnp.int32(NP - N))
    src = jnp.concatenate([edge_index[0].astype(jnp.int32), fill])
    dst = jnp.concatenate([edge_index[1].astype(jnp.int32), fill])
    zeros2d = jnp.zeros((NP, D), jnp.float32)
    zeros1d = jnp.zeros((NP,), jnp.float32)
    ones_blk = jnp.ones((BLK,), jnp.float32)

    deg0, deg1 = _sc_deg(dst, ones_blk, zeros1d)
    d0 = deg0.reshape(NP, 1)
    d1 = deg1.reshape(NP, 1)
    y = _tc_scale_in(d0, d1, x_pad)
    p1a, p1b = _sc_hop(y, zeros2d, src, dst)
    h1 = _tc_mid(d0, d1, p1a, p1b, y)
    p2a, p2b = _sc_hop(h1, zeros2d, src, dst)
    out = _tc_final(d0, d1, p2a, p2b, h1, W, b.reshape(1, D))
    return out[:N]


# speedup vs baseline: 30.1179x; 1.9128x over previous
import functools

import jax
import jax.numpy as jnp
from jax import lax
from jax.experimental import pallas as pl
from jax.experimental.pallas import tpu as pltpu
from jax.experimental.pallas import tpu_sc as plsc

N = 10000
E = 320000
D = 128

NTILE = 32
NP = 10240
EP = 327680
EPW = EP // NTILE
BLK = 128
NBLK = EPW // BLK
RPT = NP // 16

_MESH = plsc.VectorSubcoreMesh(core_axis_name="c", subcore_axis_name="s")


def _sc_deg_body(dst_ref, ones_ref, zeros_ref, deg0_ref, deg1_ref,
                 iv0, iv1, ones_v, deg_sh, is0, is1):
    c = lax.axis_index("c")
    s = lax.axis_index("s")
    iv = (iv0, iv1)
    isem = (is0, is1)
    pltpu.sync_copy(zeros_ref.at[pl.ds(s * RPT, RPT)],
                    deg_sh.at[pl.ds(s * RPT, RPT)])
    pltpu.sync_copy(ones_ref, ones_v)
    plsc.subcore_barrier()
    base_blk = (c * 16 + s) * NBLK

    def idx_issue(m, blk):
        pltpu.async_copy(dst_ref.at[pl.ds((base_blk + blk) * BLK, BLK)],
                         iv[m], isem[m])

    def idx_wait(m):
        pltpu.make_async_copy(dst_ref.at[pl.ds(0, BLK)], iv[m],
                              isem[m]).wait()

    idx_issue(0, 0)
    idx_issue(1, 1)

    def step(j, carry):
        for t in range(2):
            i = 2 * j + t
            idx_wait(t)
            pltpu.sync_copy(ones_v, deg_sh.at[iv[t]], add=True)

            @pl.when(j < NBLK // 2 - 1)
            def _():
                idx_issue(t, i + 2)

        return carry

    lax.fori_loop(0, NBLK // 2, step, 0)
    plsc.subcore_barrier()

    @pl.when(c == 0)
    def _():
        pltpu.sync_copy(deg_sh.at[pl.ds(s * RPT, RPT)],
                        deg0_ref.at[pl.ds(s * RPT, RPT)])

    @pl.when(c == 1)
    def _():
        pltpu.sync_copy(deg_sh.at[pl.ds(s * RPT, RPT)],
                        deg1_ref.at[pl.ds(s * RPT, RPT)])


_sc_deg = pl.kernel(
    _sc_deg_body,
    out_type=(jax.ShapeDtypeStruct((NP,), jnp.float32),
              jax.ShapeDtypeStruct((NP,), jnp.float32)),
    mesh=_MESH,
    scratch_types=[
        pltpu.VMEM((BLK,), jnp.int32),
        pltpu.VMEM((BLK,), jnp.int32),
        pltpu.VMEM((BLK,), jnp.float32),
        pltpu.VMEM_SHARED((NP,), jnp.float32),
        pltpu.SemaphoreType.DMA,
        pltpu.SemaphoreType.DMA,
    ],
)


def _sc_hop_body(h_ref, zeros_ref, src_ref, dst_ref, pa_ref, pb_ref,
                 sv0, sv1, sv2, sv3, sv4, sv5, sv6, sv7,
                 dv0, dv1, dv2, dv3, dv4, dv5, dv6, dv7,
                 rows0, rows1, acc_sh,
                 is0, is1, is2, is3, is4, is5, is6, is7, gs0, gs1):
    c = lax.axis_index("c")
    s = lax.axis_index("s")
    src_v = (sv0, sv1, sv2, sv3, sv4, sv5, sv6, sv7)
    dst_v = (dv0, dv1, dv2, dv3, dv4, dv5, dv6, dv7)
    rows = (rows0, rows1)
    isem = (is0, is1, is2, is3, is4, is5, is6, is7)
    gsem = (gs0, gs1)
    pltpu.sync_copy(zeros_ref.at[pl.ds(s * RPT, RPT)],
                    acc_sh.at[pl.ds(s * RPT, RPT)])
    plsc.subcore_barrier()
    base_blk = (c * 16 + s) * NBLK

    def idx_issue(m, blk):
        off = (base_blk + blk) * BLK
        pltpu.async_copy(src_ref.at[pl.ds(off, BLK)], src_v[m], isem[m])
        pltpu.async_copy(dst_ref.at[pl.ds(off, BLK)], dst_v[m], isem[m])

    def idx_wait(m):
        pltpu.make_async_copy(src_ref.at[pl.ds(0, BLK)], src_v[m],
                              isem[m]).wait()
        pltpu.make_async_copy(dst_ref.at[pl.ds(0, BLK)], dst_v[m],
                              isem[m]).wait()

    def gather_issue(k, m):
        pltpu.async_copy(h_ref.at[src_v[m]], rows[k], gsem[k])

    def gather_wait(k, m):
        pltpu.make_async_copy(h_ref.at[src_v[m]], rows[k], gsem[k]).wait()

    for t in range(8):
        idx_issue(t, t)
    for t in range(2):
        idx_wait(t)
        gather_issue(t, t)

    def step(j, carry):
        for t in range(8):
            i = 8 * j + t
            k = t % 2
            m2 = (t + 2) % 8
            gather_wait(k, t)
            pltpu.sync_copy(rows[k], acc_sh.at[dst_v[t]], add=True)

            @pl.when(j < NBLK // 8 - 1)
            def _():
                idx_issue(t, i + 8)

            if t < 6:
                idx_wait(m2)
                gather_issue(k, m2)
            else:
                @pl.when(j < NBLK // 8 - 1)
                def _():
                    idx_wait(m2)
                    gather_issue(k, m2)

        return carry

    lax.fori_loop(0, NBLK // 8, step, 0)
    plsc.subcore_barrier()

    @pl.when(c == 0)
    def _():
        pltpu.sync_copy(acc_sh.at[pl.ds(s * RPT, RPT)],
                        pa_ref.at[pl.ds(s * RPT, RPT)])

    @pl.when(c == 1)
    def _():
        pltpu.sync_copy(acc_sh.at[pl.ds(s * RPT, RPT)],
                        pb_ref.at[pl.ds(s * RPT, RPT)])


_sc_hop = pl.kernel(
    _sc_hop_body,
    out_type=(jax.ShapeDtypeStruct((NP, D), jnp.float32),
              jax.ShapeDtypeStruct((NP, D), jnp.float32)),
    mesh=_MESH,
    scratch_types=(
        [pltpu.VMEM((BLK,), jnp.int32)] * 16
        + [pltpu.VMEM((BLK, D), jnp.float32)] * 2
        + [pltpu.VMEM_SHARED((NP, D), jnp.float32)]
        + [pltpu.SemaphoreType.DMA] * 10
    ),
)


R = 1024
G = NP // R


def _tc_scale_in_body(d0_ref, d1_ref, x_ref, y_ref):
    deg = d0_ref[...] + d1_ref[...] + 1.0
    y_ref[...] = x_ref[...] * lax.rsqrt(deg)


_tc_scale_in = pl.pallas_call(
    _tc_scale_in_body,
    grid=(G,),
    in_specs=[pl.BlockSpec((R, 1), lambda i: (i, 0)),
              pl.BlockSpec((R, 1), lambda i: (i, 0)),
              pl.BlockSpec((R, D), lambda i: (i, 0))],
    out_specs=pl.BlockSpec((R, D), lambda i: (i, 0)),
    out_shape=jax.ShapeDtypeStruct((NP, D), jnp.float32),
)


def _tc_mid_body(d0_ref, d1_ref, pa_ref, pb_ref, y_ref, h1_ref):
    deg = d0_ref[...] + d1_ref[...] + 1.0
    h1_ref[...] = (pa_ref[...] + pb_ref[...] + y_ref[...]) / deg


_tc_mid = pl.pallas_call(
    _tc_mid_body,
    grid=(G,),
    in_specs=[pl.BlockSpec((R, 1), lambda i: (i, 0)),
              pl.BlockSpec((R, 1), lambda i: (i, 0)),
              pl.BlockSpec((R, D), lambda i: (i, 0)),
              pl.BlockSpec((R, D), lambda i: (i, 0)),
              pl.BlockSpec((R, D), lambda i: (i, 0))],
    out_specs=pl.BlockSpec((R, D), lambda i: (i, 0)),
    out_shape=jax.ShapeDtypeStruct((NP, D), jnp.float32),
)


def _tc_final_body(d0_ref, d1_ref, pa_ref, pb_ref, h1_ref, w_ref, b_ref, o_ref):
    deg = d0_ref[...] + d1_ref[...] + 1.0
    h2 = (pa_ref[...] + pb_ref[...] + h1_ref[...]) * lax.rsqrt(deg)
    o = jnp.dot(h2, w_ref[...], preferred_element_type=jnp.float32) + b_ref[...]
    m = jnp.max(o, axis=-1, keepdims=True)
    e = o - m
    o_ref[...] = e - jnp.log(jnp.sum(jnp.exp(e), axis=-1, keepdims=True))


_tc_final = pl.pallas_call(
    _tc_final_body,
    grid=(G,),
    in_specs=[pl.BlockSpec((R, 1), lambda i: (i, 0)),
              pl.BlockSpec((R, 1), lambda i: (i, 0)),
              pl.BlockSpec((R, D), lambda i: (i, 0)),
              pl.BlockSpec((R, D), lambda i: (i, 0)),
              pl.BlockSpec((R, D), lambda i: (i, 0)),
              pl.BlockSpec((D, D), lambda i: (0, 0)),
              pl.BlockSpec((1, D), lambda i: (0, 0))],
    out_specs=pl.BlockSpec((R, D), lambda i: (i, 0)),
    out_shape=jax.ShapeDtypeStruct((NP, D), jnp.float32),
)


def kernel(x, edge_index, W, b):
    x_pad = jnp.pad(x, ((0, NP - N), (0, 0)))
    fill = N + jax.lax.rem(jnp.arange(EP - E, dtype=jnp.int32),
                           jnp.int32(NP - N))
    src = jnp.concatenate([edge_index[0].astype(jnp.int32), fill])
    dst = jnp.concatenate([edge_index[1].astype(jnp.int32), fill])
    zeros2d = jnp.zeros((NP, D), jnp.float32)
    zeros1d = jnp.zeros((NP,), jnp.float32)
    ones_blk = jnp.ones((BLK,), jnp.float32)

    deg0, deg1 = _sc_deg(dst, ones_blk, zeros1d)
    d0 = deg0.reshape(NP, 1)
    d1 = deg1.reshape(NP, 1)
    y = _tc_scale_in(d0, d1, x_pad)
    p1a, p1b = _sc_hop(y, zeros2d, src, dst)
    h1 = _tc_mid(d0, d1, p1a, p1b, y)
    p2a, p2b = _sc_hop(h1, zeros2d, src, dst)
    out = _tc_final(d0, d1, p2a, p2b, h1, W, b.reshape(1, D))
    return out[:N]


# speedup vs baseline: 31.2714x; 1.0383x over previous
import jax
import jax.numpy as jnp
from jax import lax
from jax.experimental import pallas as pl
from jax.experimental.pallas import tpu as pltpu
from jax.experimental.pallas import tpu_sc as plsc

N = 10000
E = 320000
D = 128

NTILE = 32
NP = 10240
EP = 327680
EPW = EP // NTILE
BLK = 128
NBLK = EPW // BLK
RPT = NP // 16

_MESH = plsc.VectorSubcoreMesh(core_axis_name="c", subcore_axis_name="s")


def _sc_deg_body(dst_ref, ones_ref, zeros_ref, deg0_ref, deg1_ref,
                 iv0, iv1, ones_v, deg_sh, is0, is1):
    c = lax.axis_index("c")
    s = lax.axis_index("s")
    iv = (iv0, iv1)
    isem = (is0, is1)
    pltpu.sync_copy(zeros_ref.at[pl.ds(s * RPT, RPT)],
                    deg_sh.at[pl.ds(s * RPT, RPT)])
    pltpu.sync_copy(ones_ref, ones_v)
    plsc.subcore_barrier()
    base_blk = (c * 16 + s) * NBLK

    def idx_issue(m, blk):
        pltpu.async_copy(dst_ref.at[pl.ds((base_blk + blk) * BLK, BLK)],
                         iv[m], isem[m])

    def idx_wait(m):
        pltpu.make_async_copy(dst_ref.at[pl.ds(0, BLK)], iv[m],
                              isem[m]).wait()

    idx_issue(0, 0)
    idx_issue(1, 1)

    def step(j, carry):
        for t in range(2):
            i = 2 * j + t
            idx_wait(t)
            pltpu.sync_copy(ones_v, deg_sh.at[iv[t]], add=True)

            @pl.when(j < NBLK // 2 - 1)
            def _():
                idx_issue(t, i + 2)

        return carry

    lax.fori_loop(0, NBLK // 2, step, 0)
    plsc.subcore_barrier()

    @pl.when(c == 0)
    def _():
        pltpu.sync_copy(deg_sh.at[pl.ds(s * RPT, RPT)],
                        deg0_ref.at[pl.ds(s * RPT, RPT)])

    @pl.when(c == 1)
    def _():
        pltpu.sync_copy(deg_sh.at[pl.ds(s * RPT, RPT)],
                        deg1_ref.at[pl.ds(s * RPT, RPT)])


_sc_deg = pl.kernel(
    _sc_deg_body,
    out_type=(jax.ShapeDtypeStruct((NP,), jnp.float32),
              jax.ShapeDtypeStruct((NP,), jnp.float32)),
    mesh=_MESH,
    scratch_types=[
        pltpu.VMEM((BLK,), jnp.int32),
        pltpu.VMEM((BLK,), jnp.int32),
        pltpu.VMEM((BLK,), jnp.float32),
        pltpu.VMEM_SHARED((NP,), jnp.float32),
        pltpu.SemaphoreType.DMA,
        pltpu.SemaphoreType.DMA,
    ],
)


def _sc_hop_body(h_ref, zeros_ref, src_ref, dst_ref, pa_ref, pb_ref,
                 sv0, sv1, sv2, sv3, sv4, sv5, sv6, sv7,
                 dv0, dv1, dv2, dv3, dv4, dv5, dv6, dv7,
                 rows0, rows1, acc_sh,
                 is0, is1, is2, is3, is4, is5, is6, is7, gs0, gs1):
    c = lax.axis_index("c")
    s = lax.axis_index("s")
    src_v = (sv0, sv1, sv2, sv3, sv4, sv5, sv6, sv7)
    dst_v = (dv0, dv1, dv2, dv3, dv4, dv5, dv6, dv7)
    rows = (rows0, rows1)
    isem = (is0, is1, is2, is3, is4, is5, is6, is7)
    gsem = (gs0, gs1)
    pltpu.sync_copy(zeros_ref.at[pl.ds(s * RPT, RPT)],
                    acc_sh.at[pl.ds(s * RPT, RPT)])
    plsc.subcore_barrier()
    base_blk = (c * 16 + s) * NBLK

    def idx_issue(m, blk):
        off = (base_blk + blk) * BLK
        pltpu.async_copy(src_ref.at[pl.ds(off, BLK)], src_v[m], isem[m])
        pltpu.async_copy(dst_ref.at[pl.ds(off, BLK)], dst_v[m], isem[m])

    def idx_wait(m):
        pltpu.make_async_copy(src_ref.at[pl.ds(0, BLK)], src_v[m],
                              isem[m]).wait()
        pltpu.make_async_copy(dst_ref.at[pl.ds(0, BLK)], dst_v[m],
                              isem[m]).wait()

    def gather_issue(k, m):
        pltpu.async_copy(h_ref.at[src_v[m]], rows[k], gsem[k])

    def gather_wait(k, m):
        pltpu.make_async_copy(h_ref.at[src_v[m]], rows[k], gsem[k]).wait()

    for t in range(8):
        idx_issue(t, t)
    for t in range(2):
        idx_wait(t)
        gather_issue(t, t)

    def step(j, carry):
        for t in range(8):
            i = 8 * j + t
            k = t % 2
            m2 = (t + 2) % 8
            gather_wait(k, t)
            pltpu.sync_copy(rows[k], acc_sh.at[dst_v[t]], add=True)

            @pl.when(j < NBLK // 8 - 1)
            def _():
                idx_issue(t, i + 8)

            if t < 6:
                idx_wait(m2)
                gather_issue(k, m2)
            else:
                @pl.when(j < NBLK // 8 - 1)
                def _():
                    idx_wait(m2)
                    gather_issue(k, m2)

        return carry

    lax.fori_loop(0, NBLK // 8, step, 0)
    plsc.subcore_barrier()

    @pl.when(c == 0)
    def _():
        pltpu.sync_copy(acc_sh.at[pl.ds(s * RPT, RPT)],
                        pa_ref.at[pl.ds(s * RPT, RPT)])

    @pl.when(c == 1)
    def _():
        pltpu.sync_copy(acc_sh.at[pl.ds(s * RPT, RPT)],
                        pb_ref.at[pl.ds(s * RPT, RPT)])


_sc_hop = pl.kernel(
    _sc_hop_body,
    out_type=(jax.ShapeDtypeStruct((NP, D), jnp.float32),
              jax.ShapeDtypeStruct((NP, D), jnp.float32)),
    mesh=_MESH,
    scratch_types=(
        [pltpu.VMEM((BLK,), jnp.int32)] * 16
        + [pltpu.VMEM((BLK, D), jnp.float32)] * 2
        + [pltpu.VMEM_SHARED((NP, D), jnp.float32)]
        + [pltpu.SemaphoreType.DMA] * 10
    ),
)


R = 2048
G = NP // R


def _tc_scale_in_body(d0_ref, d1_ref, x_ref, y_ref):
    deg = d0_ref[...] + d1_ref[...] + 1.0
    y_ref[...] = x_ref[...] * lax.rsqrt(deg)


_tc_scale_in = pl.pallas_call(
    _tc_scale_in_body,
    grid=(G,),
    in_specs=[pl.BlockSpec((R, 1), lambda i: (i, 0)),
              pl.BlockSpec((R, 1), lambda i: (i, 0)),
              pl.BlockSpec((R, D), lambda i: (i, 0))],
    out_specs=pl.BlockSpec((R, D), lambda i: (i, 0)),
    out_shape=jax.ShapeDtypeStruct((NP, D), jnp.float32),
)


def _tc_mid_body(d0_ref, d1_ref, pa_ref, pb_ref, y_ref, h1_ref):
    deg = d0_ref[...] + d1_ref[...] + 1.0
    h1_ref[...] = (pa_ref[...] + pb_ref[...] + y_ref[...]) / deg


_tc_mid = pl.pallas_call(
    _tc_mid_body,
    grid=(G,),
    in_specs=[pl.BlockSpec((R, 1), lambda i: (i, 0)),
              pl.BlockSpec((R, 1), lambda i: (i, 0)),
              pl.BlockSpec((R, D), lambda i: (i, 0)),
              pl.BlockSpec((R, D), lambda i: (i, 0)),
              pl.BlockSpec((R, D), lambda i: (i, 0))],
    out_specs=pl.BlockSpec((R, D), lambda i: (i, 0)),
    out_shape=jax.ShapeDtypeStruct((NP, D), jnp.float32),
)


RF = 2000
GF = N // RF


def _tc_final_body(d0_ref, d1_ref, pa_ref, pb_ref, h1_ref, w_ref, b_ref, o_ref):
    deg = d0_ref[...] + d1_ref[...] + 1.0
    h2 = (pa_ref[...] + pb_ref[...] + h1_ref[...]) * lax.rsqrt(deg)
    o = jnp.dot(h2, w_ref[...], preferred_element_type=jnp.float32) + b_ref[...]
    m = jnp.max(o, axis=-1, keepdims=True)
    e = o - m
    o_ref[...] = e - jnp.log(jnp.sum(jnp.exp(e), axis=-1, keepdims=True))


_tc_final = pl.pallas_call(
    _tc_final_body,
    grid=(GF,),
    in_specs=[pl.BlockSpec((RF, 1), lambda i: (i, 0)),
              pl.BlockSpec((RF, 1), lambda i: (i, 0)),
              pl.BlockSpec((RF, D), lambda i: (i, 0)),
              pl.BlockSpec((RF, D), lambda i: (i, 0)),
              pl.BlockSpec((RF, D), lambda i: (i, 0)),
              pl.BlockSpec((D, D), lambda i: (0, 0)),
              pl.BlockSpec((1, D), lambda i: (0, 0))],
    out_specs=pl.BlockSpec((RF, D), lambda i: (i, 0)),
    out_shape=jax.ShapeDtypeStruct((N, D), jnp.float32),
)


def kernel(x, edge_index, W, b):
    x_pad = jnp.pad(x, ((0, NP - N), (0, 0)))
    fill = N + jax.lax.rem(jnp.arange(EP - E, dtype=jnp.int32),
                           jnp.int32(NP - N))
    src = jnp.concatenate([edge_index[0].astype(jnp.int32), fill])
    dst = jnp.concatenate([edge_index[1].astype(jnp.int32), fill])
    zeros2d = jnp.zeros((NP, D), jnp.float32)
    zeros1d = jnp.zeros((NP,), jnp.float32)
    ones_blk = jnp.ones((BLK,), jnp.float32)

    deg0, deg1 = _sc_deg(dst, ones_blk, zeros1d)
    d0 = deg0.reshape(NP, 1)
    d1 = deg1.reshape(NP, 1)
    y = _tc_scale_in(d0, d1, x_pad)
    p1a, p1b = _sc_hop(y, zeros2d, src, dst)
    h1 = _tc_mid(d0, d1, p1a, p1b, y)
    p2a, p2b = _sc_hop(h1, zeros2d, src, dst)
    return _tc_final(d0, d1, p2a, p2b, h1, W, b.reshape(1, D))
